# baseline (device time: 39067 ns/iter reference)
import jax
import jax.numpy as jnp
from jax import lax
from jax.experimental import pallas as pl
from jax.experimental.pallas import tpu as pltpu


def kernel(x, W):
    m, d = x.shape
    _, v_half = W.shape
    v = 2 * v_half

    def body(x_ref, w_ref, out_ref, comm_ref, send_sem, recv_sem):
        my_x = lax.axis_index("x")
        my_y = lax.axis_index("y")
        my_z = lax.axis_index("z")
        partner = (my_x, my_y, 1 - my_z)

        barrier_sem = pltpu.get_barrier_semaphore()
        pl.semaphore_signal(
            barrier_sem,
            inc=1,
            device_id=partner,
            device_id_type=pl.DeviceIdType.MESH,
        )
        pl.semaphore_wait(barrier_sem, 1)

        logits = jnp.dot(
            x_ref[...].astype(jnp.bfloat16),
            w_ref[...].astype(jnp.bfloat16),
            preferred_element_type=jnp.float32,
        )
        comm_ref[0, :, :] = logits.astype(jnp.bfloat16)

        rdma = pltpu.make_async_remote_copy(
            src_ref=comm_ref.at[0],
            dst_ref=comm_ref.at[1],
            send_sem=send_sem,
            recv_sem=recv_sem,
            device_id=partner,
            device_id_type=pl.DeviceIdType.MESH,
        )
        rdma.start()
        rdma.wait()

        own = comm_ref[0, :, :].astype(jnp.float32)
        oth = comm_ref[1, :, :].astype(jnp.float32)
        g = jnp.maximum(
            own.max(axis=1, keepdims=True), oth.max(axis=1, keepdims=True)
        )
        e_own = jnp.exp(own - g)
        e_oth = jnp.exp(oth - g)
        s = e_own.sum(axis=1, keepdims=True) + e_oth.sum(axis=1, keepdims=True)
        p_own = e_own / s
        p_oth = e_oth / s

        @pl.when(my_z == 0)
        def _():
            out_ref[:, :v_half] = p_own
            out_ref[:, v_half:] = p_oth

        @pl.when(my_z == 1)
        def _():
            out_ref[:, :v_half] = p_oth
            out_ref[:, v_half:] = p_own

    return pl.pallas_call(
        body,
        out_shape=jax.ShapeDtypeStruct((m, v), jnp.float32),
        in_specs=[
            pl.BlockSpec(memory_space=pltpu.VMEM),
            pl.BlockSpec(memory_space=pltpu.VMEM),
        ],
        out_specs=pl.BlockSpec(memory_space=pltpu.VMEM),
        scratch_shapes=[
            pltpu.VMEM((2, m, v_half), jnp.bfloat16),
            pltpu.SemaphoreType.DMA,
            pltpu.SemaphoreType.DMA,
        ],
        compiler_params=pltpu.CompilerParams(collective_id=0),
    )(x, W)


# device time: 35708 ns/iter; 1.0941x vs baseline; 1.0941x over previous
import jax
import jax.numpy as jnp
from jax import lax
from jax.experimental import pallas as pl
from jax.experimental.pallas import tpu as pltpu

K = 4


def kernel(x, W):
    m, d = x.shape
    _, v_half = W.shape
    v = 2 * v_half
    tile = v_half // K

    def body(x_ref, w_ref, out_ref, comm_ref, send_sems, recv_sems):
        my_x = lax.axis_index("x")
        my_y = lax.axis_index("y")
        my_z = lax.axis_index("z")
        partner = (my_x, my_y, 1 - my_z)
        own_base = my_z * v_half
        oth_base = (1 - my_z) * v_half

        barrier_sem = pltpu.get_barrier_semaphore()
        pl.semaphore_signal(
            barrier_sem,
            inc=1,
            device_id=partner,
            device_id_type=pl.DeviceIdType.MESH,
        )
        pl.semaphore_wait(barrier_sem, 1)

        def tile_rdma(k):
            return pltpu.make_async_remote_copy(
                src_ref=comm_ref.at[0, k],
                dst_ref=comm_ref.at[1, k],
                send_sem=send_sems.at[k],
                recv_sem=recv_sems.at[k],
                device_id=partner,
                device_id_type=pl.DeviceIdType.MESH,
            )

        xb = x_ref[...].astype(jnp.bfloat16)
        for k in range(K):
            lk = jnp.dot(
                xb,
                w_ref[:, k * tile : (k + 1) * tile].astype(jnp.bfloat16),
                preferred_element_type=jnp.float32,
            )
            comm_ref[0, k] = lk.astype(jnp.bfloat16)
            tile_rdma(k).start()

        own = [comm_ref[0, k].astype(jnp.float32) for k in range(K)]
        m_own = own[0].max(axis=1, keepdims=True)
        for k in range(1, K):
            m_own = jnp.maximum(m_own, own[k].max(axis=1, keepdims=True))
        s = jnp.zeros((m, 1), jnp.float32)
        for k in range(K):
            e = jnp.exp(own[k] - m_own)
            s = s + e.sum(axis=1, keepdims=True)
            out_ref[:, pl.ds(own_base + k * tile, tile)] = e

        for k in range(K):
            tile_rdma(k).wait_recv()
            e = jnp.exp(comm_ref[1, k].astype(jnp.float32) - m_own)
            s = s + e.sum(axis=1, keepdims=True)
            out_ref[:, pl.ds(oth_base + k * tile, tile)] = e

        out_ref[:, :] = out_ref[:, :] * (1.0 / s)

        for k in range(K):
            tile_rdma(k).wait_send()

    return pl.pallas_call(
        body,
        out_shape=jax.ShapeDtypeStruct((m, v), jnp.float32),
        in_specs=[
            pl.BlockSpec(memory_space=pltpu.VMEM),
            pl.BlockSpec(memory_space=pltpu.VMEM),
        ],
        out_specs=pl.BlockSpec(memory_space=pltpu.VMEM),
        scratch_shapes=[
            pltpu.VMEM((2, K, m, tile), jnp.bfloat16),
            pltpu.SemaphoreType.DMA((K,)),
            pltpu.SemaphoreType.DMA((K,)),
        ],
        compiler_params=pltpu.CompilerParams(collective_id=0),
    )(x, W)


# device time: 35529 ns/iter; 1.0996x vs baseline; 1.0050x over previous
import jax
import jax.numpy as jnp
from jax import lax
from jax.experimental import pallas as pl
from jax.experimental.pallas import tpu as pltpu

K = 4


def kernel(x, W):
    m, d = x.shape
    _, v_half = W.shape
    v = 2 * v_half
    tile = v_half // K

    def body(x_ref, w_ref, out_ref, comm_ref, send_sems, recv_sems):
        my_x = lax.axis_index("x")
        my_y = lax.axis_index("y")
        my_z = lax.axis_index("z")
        partner = (my_x, my_y, 1 - my_z)

        barrier_sem = pltpu.get_barrier_semaphore()
        pl.semaphore_signal(
            barrier_sem,
            inc=1,
            device_id=partner,
            device_id_type=pl.DeviceIdType.MESH,
        )
        pl.semaphore_wait(barrier_sem, 1)

        def tile_rdma(k):
            return pltpu.make_async_remote_copy(
                src_ref=comm_ref.at[0, k],
                dst_ref=comm_ref.at[1, k],
                send_sem=send_sems.at[k],
                recv_sem=recv_sems.at[k],
                device_id=partner,
                device_id_type=pl.DeviceIdType.MESH,
            )

        xb = x_ref[...].astype(jnp.bfloat16)
        for k in range(K):
            lk = jnp.dot(
                xb,
                w_ref[:, k * tile : (k + 1) * tile].astype(jnp.bfloat16),
                preferred_element_type=jnp.float32,
            )
            comm_ref[0, k] = lk.astype(jnp.bfloat16)
            tile_rdma(k).start()

        def softmax_halves(own_base, oth_base):
            own = [comm_ref[0, k].astype(jnp.float32) for k in range(K)]
            m_own = own[0].max(axis=1, keepdims=True)
            for k in range(1, K):
                m_own = jnp.maximum(m_own, own[k].max(axis=1, keepdims=True))
            s = jnp.zeros((m, 1), jnp.float32)
            for k in range(K):
                e = jnp.exp(own[k] - m_own)
                s = s + e.sum(axis=1, keepdims=True)
                out_ref[:, own_base + k * tile : own_base + (k + 1) * tile] = e

            for k in range(K):
                tile_rdma(k).wait_recv()
                e = jnp.exp(comm_ref[1, k].astype(jnp.float32) - m_own)
                s = s + e.sum(axis=1, keepdims=True)
                out_ref[:, oth_base + k * tile : oth_base + (k + 1) * tile] = e

            out_ref[:, :] = out_ref[:, :] * (1.0 / s)

        @pl.when(my_z == 0)
        def _():
            softmax_halves(0, v_half)

        @pl.when(my_z == 1)
        def _():
            softmax_halves(v_half, 0)

        for k in range(K):
            tile_rdma(k).wait_send()

    return pl.pallas_call(
        body,
        out_shape=jax.ShapeDtypeStruct((m, v), jnp.float32),
        in_specs=[
            pl.BlockSpec(memory_space=pltpu.VMEM),
            pl.BlockSpec(memory_space=pltpu.VMEM),
        ],
        out_specs=pl.BlockSpec(memory_space=pltpu.VMEM),
        scratch_shapes=[
            pltpu.VMEM((2, K, m, tile), jnp.bfloat16),
            pltpu.SemaphoreType.DMA((K,)),
            pltpu.SemaphoreType.DMA((K,)),
        ],
        compiler_params=pltpu.CompilerParams(collective_id=0),
    )(x, W)


# device time: 26235 ns/iter; 1.4891x vs baseline; 1.3543x over previous
import os

import jax
import jax.numpy as jnp
from jax import lax
from jax.experimental import pallas as pl
from jax.experimental.pallas import tpu as pltpu

NQ = 4
S = int(os.environ.get("BENCH_S", "4"))
SLANES = 8


def kernel(x, W):
    m, d = x.shape
    _, v_half = W.shape
    v = 2 * v_half
    tile = v_half // NQ
    sub = tile // S
    hw = sub // 2

    def body(
        x_ref,
        w_ref,
        out_ref,
        ebuf,
        crossbuf,
        sbuf,
        zsend_sems,
        zrecv_sems,
        stat_sems,
        psend_sems,
        precv_sems,
        fsend_sems,
        frecv_sems,
    ):
        mx = lax.axis_index("x")
        my = lax.axis_index("y")
        mz = lax.axis_index("z")
        zpartner = (mx, my, 1 - mz)
        xpeer = (1 - mx, my, mz)
        ypeer = (mx, 1 - my, mz)

        q = 2 * mx + my
        q_x = 2 * (1 - mx) + my
        q_y = 2 * mx + (1 - my)
        q_d = 2 * (1 - mx) + (1 - my)
        own_base = mz * v_half
        oth_base = (1 - mz) * v_half

        barrier_sem = pltpu.get_barrier_semaphore()
        for peer in (zpartner, xpeer, ypeer):
            pl.semaphore_signal(
                barrier_sem,
                inc=1,
                device_id=peer,
                device_id_type=pl.DeviceIdType.MESH,
            )
        pl.semaphore_wait(barrier_sem, 3)

        def z_rdma(s):
            return pltpu.make_async_remote_copy(
                src_ref=ebuf.at[q, s],
                dst_ref=crossbuf.at[q, s],
                send_sem=zsend_sems.at[s],
                recv_sem=zrecv_sems.at[s],
                device_id=zpartner,
                device_id_type=pl.DeviceIdType.MESH,
            )

        stat_rdma = pltpu.make_async_remote_copy(
            src_ref=sbuf.at[0],
            dst_ref=sbuf.at[1],
            send_sem=stat_sems.at[0],
            recv_sem=stat_sems.at[1],
            device_id=zpartner,
            device_id_type=pl.DeviceIdType.MESH,
        )

        def direct_rdma(role, s, peer):
            return pltpu.make_async_remote_copy(
                src_ref=crossbuf.at[q, s],
                dst_ref=crossbuf.at[q, s],
                send_sem=psend_sems.at[role, s],
                recv_sem=precv_sems.at[role, s],
                device_id=peer,
                device_id_type=pl.DeviceIdType.MESH,
            )

        def fwd_to_y(s):
            return pltpu.make_async_remote_copy(
                src_ref=crossbuf.at[q_x, s, 0],
                dst_ref=crossbuf.at[q_x, s, 0],
                send_sem=fsend_sems.at[0, s],
                recv_sem=frecv_sems.at[1, s],
                device_id=ypeer,
                device_id_type=pl.DeviceIdType.MESH,
            )

        def fwd_to_x(s):
            return pltpu.make_async_remote_copy(
                src_ref=crossbuf.at[q_y, s, 1],
                dst_ref=crossbuf.at[q_y, s, 1],
                send_sem=fsend_sems.at[1, s],
                recv_sem=frecv_sems.at[0, s],
                device_id=xpeer,
                device_id_type=pl.DeviceIdType.MESH,
            )

        def store_halves(ref, slot, s):
            return jnp.concatenate(
                [
                    ref[slot, s, 0].astype(jnp.float32),
                    ref[slot, s, 1].astype(jnp.float32),
                ],
                axis=1,
            )

        xb = x_ref[...].astype(jnp.bfloat16)
        s_own = jnp.zeros((m, 1), jnp.float32)
        for ss in range(S):
            l = jnp.dot(
                xb,
                w_ref[:, pl.ds(q * tile + ss * sub, sub)].astype(jnp.bfloat16),
                preferred_element_type=jnp.float32,
            )
            e = jnp.exp(l)
            eb = e.astype(jnp.bfloat16)
            ebuf[q, ss, 0] = eb[:, :hw]
            ebuf[q, ss, 1] = eb[:, hw:]
            s_own = s_own + e.sum(axis=1, keepdims=True)
            z_rdma(ss).start()
        for j in range(1, NQ):
            slot = (q + j) % NQ
            l = jnp.dot(
                xb,
                w_ref[:, pl.ds(slot * tile, tile)].astype(jnp.bfloat16),
                preferred_element_type=jnp.float32,
            )
            e = jnp.exp(l)
            eb = e.astype(jnp.bfloat16)
            for ss in range(S):
                ebuf[slot, ss, 0] = eb[:, ss * sub : ss * sub + hw]
                ebuf[slot, ss, 1] = eb[:, ss * sub + hw : (ss + 1) * sub]
            s_own = s_own + e.sum(axis=1, keepdims=True)

        sbuf[0] = jnp.broadcast_to(s_own, (m, SLANES))
        stat_rdma.start()

        for ss in range(S):
            z_rdma(ss).wait_recv()
            direct_rdma(0, ss, xpeer).start()
            direct_rdma(1, ss, ypeer).start()

        stat_rdma.wait_recv()
        inv = 1.0 / (s_own + sbuf[1, :, 0:1])

        for j in range(NQ):
            slot = (q + j) % NQ
            for ss in range(S):
                out_ref[:, pl.ds(own_base + slot * tile + ss * sub, sub)] = (
                    store_halves(ebuf, slot, ss) * inv
                )
        for ss in range(S):
            out_ref[:, pl.ds(oth_base + q * tile + ss * sub, sub)] = (
                store_halves(crossbuf, q, ss) * inv
            )

        for ss in range(S):
            direct_rdma(0, ss, xpeer).wait_recv()
            fwd_to_y(ss).start()
            out_ref[:, pl.ds(oth_base + q_x * tile + ss * sub, sub)] = (
                store_halves(crossbuf, q_x, ss) * inv
            )
            direct_rdma(1, ss, ypeer).wait_recv()
            fwd_to_x(ss).start()
            out_ref[:, pl.ds(oth_base + q_y * tile + ss * sub, sub)] = (
                store_halves(crossbuf, q_y, ss) * inv
            )

        for ss in range(S):
            pltpu.make_async_remote_copy(
                src_ref=crossbuf.at[q_d, ss, 1],
                dst_ref=crossbuf.at[q_d, ss, 1],
                send_sem=fsend_sems.at[1, ss],
                recv_sem=frecv_sems.at[0, ss],
                device_id=xpeer,
                device_id_type=pl.DeviceIdType.MESH,
            ).wait_recv()
            pltpu.make_async_remote_copy(
                src_ref=crossbuf.at[q_d, ss, 0],
                dst_ref=crossbuf.at[q_d, ss, 0],
                send_sem=fsend_sems.at[0, ss],
                recv_sem=frecv_sems.at[1, ss],
                device_id=ypeer,
                device_id_type=pl.DeviceIdType.MESH,
            ).wait_recv()
            out_ref[:, pl.ds(oth_base + q_d * tile + ss * sub, sub)] = (
                store_halves(crossbuf, q_d, ss) * inv
            )

        for ss in range(S):
            z_rdma(ss).wait_send()
        stat_rdma.wait_send()
        for ss in range(S):
            direct_rdma(0, ss, xpeer).wait_send()
            direct_rdma(1, ss, ypeer).wait_send()
            fwd_to_y(ss).wait_send()
            fwd_to_x(ss).wait_send()

    return pl.pallas_call(
        body,
        out_shape=jax.ShapeDtypeStruct((m, v), jnp.float32),
        in_specs=[
            pl.BlockSpec(memory_space=pltpu.VMEM),
            pl.BlockSpec(memory_space=pltpu.VMEM),
        ],
        out_specs=pl.BlockSpec(memory_space=pltpu.VMEM),
        scratch_shapes=[
            pltpu.VMEM((NQ, S, 2, m, hw), jnp.bfloat16),
            pltpu.VMEM((NQ, S, 2, m, hw), jnp.bfloat16),
            pltpu.VMEM((2, m, SLANES), jnp.float32),
            pltpu.SemaphoreType.DMA((S,)),
            pltpu.SemaphoreType.DMA((S,)),
            pltpu.SemaphoreType.DMA((2,)),
            pltpu.SemaphoreType.DMA((2, S)),
            pltpu.SemaphoreType.DMA((2, S)),
            pltpu.SemaphoreType.DMA((2, S)),
            pltpu.SemaphoreType.DMA((2, S)),
        ],
        compiler_params=pltpu.CompilerParams(collective_id=0),
    )(x, W)
